# baseline (speedup 1.0000x reference)
import functools

import jax
import jax.numpy as jnp
from jax import lax
from jax.experimental import pallas as pl
from jax.experimental.pallas import tpu as pltpu
from jax.experimental.pallas import tpu_sc as plsc

N = 10000
NFEAT = 128
NHID = 256
NCLASS = 40
NCLS_PAD = 64
EPS = 1e-5

NC = 2
NS = 16
GROUP = 128
ACC_ROWS = 10240
ZR = ACC_ROWS // NS
ZR_MAIN = N - (NS - 1) * ZR
CHUNK = 32


def _make_sc_aggregate(d: int, g: int):
    mesh = plsc.VectorSubcoreMesh(core_axis_name="c", subcore_axis_name="s")

    nbuf = 2 if d == 128 else 4
    stage_tab = d * (N + ACC_ROWS) * 4 <= 6 * 2**20
    tab_rows = N // NS

    @functools.partial(
        pl.kernel,
        out_type=jax.ShapeDtypeStruct((NC, N, d), jnp.float32),
        mesh=mesh,
        compiler_params=pltpu.CompilerParams(use_tc_tiling_on_sc=False),
        scratch_types=(
            [pltpu.VMEM((CHUNK, GROUP), jnp.int32),
             pltpu.VMEM((CHUNK, GROUP), jnp.int32)]
            + [pltpu.VMEM((GROUP, d), jnp.float32) for _ in range(nbuf)]
            + [pltpu.VMEM_SHARED((ACC_ROWS, d), jnp.float32)]
            + ([pltpu.VMEM_SHARED((N, d), jnp.float32)] if stage_tab else [])
            + [pltpu.SemaphoreType.DMA for _ in range(nbuf)]
        ),
    )
    def agg(table_hbm, src_r, dst_r, out, src_v, dst_v, *rest):
        rows = rest[:nbuf]
        acc = rest[nbuf]
        if stage_tab:
            table = rest[nbuf + 1]
            sems = rest[nbuf + 2:]
        else:
            table = table_hbm
            sems = rest[nbuf + 1:]
        c = lax.axis_index("c")
        s = lax.axis_index("s")
        w = s * NC + c

        if stage_tab:
            pltpu.sync_copy(table_hbm.at[pl.ds(s * tab_rows, tab_rows)],
                            table.at[pl.ds(s * tab_rows, tab_rows)])

        def zbody(i, carry):
            for k in range(d // 16):
                rows[0][i, pl.ds(16 * k, 16)] = jnp.zeros((16,), jnp.float32)
            return carry

        lax.fori_loop(0, GROUP, zbody, 0)
        for r in range(ZR // GROUP):
            pltpu.sync_copy(rows[0], acc.at[pl.ds(s * ZR + r * GROUP, GROUP)])
        plsc.subcore_barrier()

        def run(gbase, gw):
            for off in range(0, gw, CHUNK):
                cs = min(CHUNK, gw - off)
                pltpu.sync_copy(src_r.at[pl.ds(gbase + off, cs)],
                                src_v.at[pl.ds(0, cs)])
                pltpu.sync_copy(dst_r.at[pl.ds(gbase + off, cs)],
                                dst_v.at[pl.ds(0, cs)])
                for k in range(min(nbuf, cs)):
                    pltpu.async_copy(table.at[src_v.at[k]], rows[k], sems[k])
                nfull = cs // nbuf

                def body(i, carry, cs=cs):
                    j0 = i * nbuf
                    for k in range(nbuf):
                        j = j0 + k
                        pltpu.make_async_copy(table.at[src_v.at[0]], rows[k],
                                              sems[k]).wait()
                        pltpu.sync_copy(rows[k], acc.at[dst_v.at[j]],
                                        add=True)

                        @pl.when(j + nbuf < cs)
                        def _(k=k, j=j):
                            pltpu.async_copy(table.at[src_v.at[j + nbuf]],
                                             rows[k], sems[k])
                    return carry

                if nfull:
                    lax.fori_loop(0, nfull, body, 0)
                for k in range(cs % nbuf):
                    j = nfull * nbuf + k
                    pltpu.make_async_copy(table.at[src_v.at[0]], rows[k],
                                          sems[k]).wait()
                    pltpu.sync_copy(rows[k], acc.at[dst_v.at[j]], add=True)

        rem = g % (NC * NS)
        gq = g // (NC * NS)
        gbase = w * gq + jnp.minimum(w, rem)
        if rem:
            @pl.when(w < rem)
            def _():
                run(gbase, gq + 1)

            @pl.when(w >= rem)
            def _():
                run(gbase, gq)
        else:
            run(gbase, gq)

        plsc.subcore_barrier()

        base = s * ZR
        pltpu.sync_copy(acc.at[pl.ds(base, ZR_MAIN)],
                        out.at[c].at[pl.ds(base, ZR_MAIN)])

        @pl.when(s < NS - 1)
        def _():
            pltpu.sync_copy(acc.at[pl.ds(base + ZR_MAIN, ZR - ZR_MAIN)],
                            out.at[c].at[pl.ds(base + ZR_MAIN, ZR - ZR_MAIN)])

    return agg


_BM = 2000


def _tca_body(ap, cs_ref, g_ref):
    a = ap[0] + ap[1]

    @pl.when(pl.program_id(0) == 0)
    def _():
        cs_ref[...] = jnp.zeros_like(cs_ref)
        g_ref[...] = jnp.zeros_like(g_ref)

    cs_ref[...] += jnp.sum(a, axis=0, keepdims=True)
    g_ref[...] += lax.dot_general(a, a, (((0,), (0,)), ((), ())),
                                  preferred_element_type=jnp.float32)


def _tcs_body(cs, g, w1, gamma, beta, scale_ref, shift_ref):
    w1v = w1[...]
    mean0 = jnp.dot(cs[...], w1v, preferred_element_type=jnp.float32) * (
        1.0 / N)
    gw = jnp.dot(g[...], w1v, preferred_element_type=jnp.float32)
    ssq = jnp.sum(w1v * gw, axis=0, keepdims=True)
    var = ssq * (1.0 / N) - mean0 * mean0
    scale = gamma[...] * lax.rsqrt(var + EPS)
    scale_ref[...] = scale
    shift_ref[...] = beta[...] - mean0 * scale


def _tcc_body(ap, w1, scale, shift, w2, y_ref):
    a = ap[0] + ap[1]
    h0 = jnp.dot(a, w1[...], preferred_element_type=jnp.float32)
    hr = jnp.maximum(h0 * scale[...] + shift[...], 0.0)
    y_ref[...] = jnp.dot(hr, w2[...], preferred_element_type=jnp.float32)


CROWS = 313
CLAST = N - 31 * CROWS


def _sc_combine(p, b2p):
    mesh = plsc.VectorSubcoreMesh(core_axis_name="c", subcore_axis_name="s")

    @functools.partial(
        pl.kernel,
        out_type=jax.ShapeDtypeStruct((N, NCLS_PAD), jnp.float32),
        mesh=mesh,
        compiler_params=pltpu.CompilerParams(use_tc_tiling_on_sc=False),
        scratch_types=[
            pltpu.VMEM((CROWS, NCLS_PAD), jnp.float32),
            pltpu.VMEM((CROWS, NCLS_PAD), jnp.float32),
            pltpu.VMEM((NCLS_PAD,), jnp.float32),
        ],
    )
    def comb(p_r, b2_r, out, buf0, buf1, bias_v):
        c = lax.axis_index("c")
        sx = lax.axis_index("s")
        w = sx * NC + c
        base = w * CROWS
        pltpu.sync_copy(b2_r, bias_v)

        def do(nrows):
            pltpu.sync_copy(p_r.at[0].at[pl.ds(base, nrows)],
                            buf0.at[pl.ds(0, nrows)])
            pltpu.sync_copy(p_r.at[1].at[pl.ds(base, nrows)],
                            buf1.at[pl.ds(0, nrows)])
            bias = [bias_v[pl.ds(16 * k, 16)]
                    for k in range(NCLS_PAD // 16)]

            def add_body(r, carry):
                for k in range(NCLS_PAD // 16):
                    col = 16 * k
                    buf0[r, pl.ds(col, 16)] = (buf0[r, pl.ds(col, 16)]
                                               + buf1[r, pl.ds(col, 16)]
                                               + bias[k])
                return carry

            lax.fori_loop(0, nrows, add_body, 0)
            pltpu.sync_copy(buf0.at[pl.ds(0, nrows)],
                            out.at[pl.ds(base, nrows)])

        @pl.when(w < 31)
        def _():
            do(CROWS)

        @pl.when(w == 31)
        def _():
            do(CLAST)

    return comb(p, b2p)


def kernel(x, edge_index, W1, b1, gamma, beta, W2, b2):
    e = edge_index.shape[1]
    g_total = -(-e // GROUP)
    e_pad = g_total * GROUP
    if e_pad == e:
        src = edge_index[0].reshape(g_total, GROUP)
        dst = edge_index[1].reshape(g_total, GROUP)
    else:
        pad_idx = jnp.arange(e_pad - e, dtype=jnp.int32)
        dummy_src = pad_idx % N
        dummy_dst = N + pad_idx % (ACC_ROWS - N)
        src = jnp.concatenate([edge_index[0], dummy_src]).reshape(
            g_total, GROUP)
        dst = jnp.concatenate([edge_index[1], dummy_dst]).reshape(
            g_total, GROUP)

    w2p = jnp.pad(W2, ((0, 0), (0, NCLS_PAD - NCLASS)))

    agg1 = _make_sc_aggregate(NFEAT, g_total)(x, src, dst)

    grid = (N // _BM,)
    cs, gmat = pl.pallas_call(
        _tca_body,
        grid=grid,
        in_specs=[
            pl.BlockSpec((NC, _BM, NFEAT), lambda i: (0, i, 0)),
        ],
        out_specs=[
            pl.BlockSpec((1, NFEAT), lambda i: (0, 0)),
            pl.BlockSpec((NFEAT, NFEAT), lambda i: (0, 0)),
        ],
        out_shape=[
            jax.ShapeDtypeStruct((1, NFEAT), jnp.float32),
            jax.ShapeDtypeStruct((NFEAT, NFEAT), jnp.float32),
        ],
    )(agg1)

    scale, shift = pl.pallas_call(
        _tcs_body,
        out_shape=[
            jax.ShapeDtypeStruct((1, NHID), jnp.float32),
            jax.ShapeDtypeStruct((1, NHID), jnp.float32),
        ],
    )(cs, gmat, W1, gamma.reshape(1, NHID), beta.reshape(1, NHID))

    y = pl.pallas_call(
        _tcc_body,
        grid=grid,
        in_specs=[
            pl.BlockSpec((NC, _BM, NFEAT), lambda i: (0, i, 0)),
            pl.BlockSpec((NFEAT, NHID), lambda i: (0, 0)),
            pl.BlockSpec((1, NHID), lambda i: (0, 0)),
            pl.BlockSpec((1, NHID), lambda i: (0, 0)),
            pl.BlockSpec((NHID, NCLS_PAD), lambda i: (0, 0)),
        ],
        out_specs=pl.BlockSpec((_BM, NCLS_PAD), lambda i: (i, 0)),
        out_shape=jax.ShapeDtypeStruct((N, NCLS_PAD), jnp.float32),
    )(agg1, W1, scale, shift, w2p)

    agg2 = _make_sc_aggregate(NCLS_PAD, g_total)(y, src, dst)

    b2p = jnp.pad(b2, (0, NCLS_PAD - NCLASS))
    out64 = _sc_combine(agg2, b2p)

    return out64[:, :NCLASS]


# speedup vs baseline: 1.1191x; 1.1191x over previous
import functools

import jax
import jax.numpy as jnp
from jax import lax
from jax.experimental import pallas as pl
from jax.experimental.pallas import tpu as pltpu
from jax.experimental.pallas import tpu_sc as plsc

N = 10000
NFEAT = 128
NHID = 256
NCLASS = 40
NCLS_PAD = 64
EPS = 1e-5

NC = 2
NS = 16
GROUP = 128
ACC_ROWS = 10240
ZR = ACC_ROWS // NS
ZR_MAIN = N - (NS - 1) * ZR
CHUNK = 32


def _make_sc_aggregate(d: int, g: int):
    mesh = plsc.VectorSubcoreMesh(core_axis_name="c", subcore_axis_name="s")

    nbuf = 2 if d == 128 else 4

    @functools.partial(
        pl.kernel,
        out_type=jax.ShapeDtypeStruct((NC, N, d), jnp.float32),
        mesh=mesh,
        compiler_params=pltpu.CompilerParams(use_tc_tiling_on_sc=(d == 128)),
        scratch_types=(
            [pltpu.VMEM((CHUNK, GROUP), jnp.int32),
             pltpu.VMEM((CHUNK, GROUP), jnp.int32)]
            + [pltpu.VMEM((GROUP, d), jnp.float32) for _ in range(nbuf)]
            + [pltpu.VMEM_SHARED((ACC_ROWS, d), jnp.float32)]
            + [pltpu.SemaphoreType.DMA for _ in range(nbuf)]
        ),
    )
    def agg(table, src_r, dst_r, out, src_v, dst_v, *rest):
        rows = rest[:nbuf]
        acc = rest[nbuf]
        sems = rest[nbuf + 1:]
        c = lax.axis_index("c")
        s = lax.axis_index("s")
        w = s * NC + c

        def zbody(i, carry):
            for k in range(d // 16):
                rows[0][i, pl.ds(16 * k, 16)] = jnp.zeros((16,), jnp.float32)
            return carry

        lax.fori_loop(0, GROUP, zbody, 0)
        for r in range(ZR // GROUP):
            pltpu.sync_copy(rows[0], acc.at[pl.ds(s * ZR + r * GROUP, GROUP)])
        plsc.subcore_barrier()

        def run(gbase, gw):
            for off in range(0, gw, CHUNK):
                cs = min(CHUNK, gw - off)
                pltpu.sync_copy(src_r.at[pl.ds(gbase + off, cs)],
                                src_v.at[pl.ds(0, cs)])
                pltpu.sync_copy(dst_r.at[pl.ds(gbase + off, cs)],
                                dst_v.at[pl.ds(0, cs)])
                for k in range(min(nbuf, cs)):
                    pltpu.async_copy(table.at[src_v.at[k]], rows[k], sems[k])
                nfull = cs // nbuf

                def body(i, carry, cs=cs):
                    j0 = i * nbuf
                    for k in range(nbuf):
                        j = j0 + k
                        pltpu.make_async_copy(table.at[src_v.at[0]], rows[k],
                                              sems[k]).wait()
                        pltpu.sync_copy(rows[k], acc.at[dst_v.at[j]],
                                        add=True)

                        @pl.when(j + nbuf < cs)
                        def _(k=k, j=j):
                            pltpu.async_copy(table.at[src_v.at[j + nbuf]],
                                             rows[k], sems[k])
                    return carry

                if nfull:
                    lax.fori_loop(0, nfull, body, 0)
                for k in range(cs % nbuf):
                    j = nfull * nbuf + k
                    pltpu.make_async_copy(table.at[src_v.at[0]], rows[k],
                                          sems[k]).wait()
                    pltpu.sync_copy(rows[k], acc.at[dst_v.at[j]], add=True)

        nw = NC * NS
        if d == 128:
            blocks = g // 8
            rem8 = blocks % nw
            bq = blocks // nw
            leftover = g % 8
            gbase = 8 * (w * bq + jnp.minimum(w, rem8))
            if rem8:
                @pl.when(w < rem8)
                def _():
                    run(gbase, 8 * (bq + 1))

                @pl.when(w >= rem8)
                def _():
                    run(gbase, 8 * bq)
            else:
                run(gbase, 8 * bq)
            if leftover:
                @pl.when(w == nw - 1)
                def _():
                    run(8 * blocks, leftover)
        else:
            rem = g % nw
            gq = g // nw
            gbase = w * gq + jnp.minimum(w, rem)
            if rem:
                @pl.when(w < rem)
                def _():
                    run(gbase, gq + 1)

                @pl.when(w >= rem)
                def _():
                    run(gbase, gq)
            else:
                run(gbase, gq)

        plsc.subcore_barrier()

        base = s * ZR
        pltpu.sync_copy(acc.at[pl.ds(base, ZR_MAIN)],
                        out.at[c].at[pl.ds(base, ZR_MAIN)])

        @pl.when(s < NS - 1)
        def _():
            pltpu.sync_copy(acc.at[pl.ds(base + ZR_MAIN, ZR - ZR_MAIN)],
                            out.at[c].at[pl.ds(base + ZR_MAIN, ZR - ZR_MAIN)])

    return agg


_BM = 2000


def _tca_body(ap, cs_ref, g_ref):
    a = ap[0] + ap[1]

    @pl.when(pl.program_id(0) == 0)
    def _():
        cs_ref[...] = jnp.zeros_like(cs_ref)
        g_ref[...] = jnp.zeros_like(g_ref)

    cs_ref[...] += jnp.sum(a, axis=0, keepdims=True)
    g_ref[...] += lax.dot_general(a, a, (((0,), (0,)), ((), ())),
                                  preferred_element_type=jnp.float32)


def _tcs_body(cs, g, w1, gamma, beta, scale_ref, shift_ref):
    w1v = w1[...]
    mean0 = jnp.dot(cs[...], w1v, preferred_element_type=jnp.float32) * (
        1.0 / N)
    gw = jnp.dot(g[...], w1v, preferred_element_type=jnp.float32)
    ssq = jnp.sum(w1v * gw, axis=0, keepdims=True)
    var = ssq * (1.0 / N) - mean0 * mean0
    scale = gamma[...] * lax.rsqrt(var + EPS)
    scale_ref[...] = scale
    shift_ref[...] = beta[...] - mean0 * scale


def _tcc_body(ap, w1, scale, shift, w2, y_ref):
    a = ap[0] + ap[1]
    h0 = jnp.dot(a, w1[...], preferred_element_type=jnp.float32)
    hr = jnp.maximum(h0 * scale[...] + shift[...], 0.0)
    y_ref[...] = jnp.dot(hr, w2[...], preferred_element_type=jnp.float32)


CROWS = 313
CLAST = N - 31 * CROWS


def _sc_combine(p, b2p):
    mesh = plsc.VectorSubcoreMesh(core_axis_name="c", subcore_axis_name="s")

    @functools.partial(
        pl.kernel,
        out_type=jax.ShapeDtypeStruct((N, NCLS_PAD), jnp.float32),
        mesh=mesh,
        compiler_params=pltpu.CompilerParams(use_tc_tiling_on_sc=False),
        scratch_types=[
            pltpu.VMEM((CROWS, NCLS_PAD), jnp.float32),
            pltpu.VMEM((CROWS, NCLS_PAD), jnp.float32),
            pltpu.VMEM((NCLS_PAD,), jnp.float32),
        ],
    )
    def comb(p_r, b2_r, out, buf0, buf1, bias_v):
        c = lax.axis_index("c")
        sx = lax.axis_index("s")
        w = sx * NC + c
        base = w * CROWS
        pltpu.sync_copy(b2_r, bias_v)

        def do(nrows):
            pltpu.sync_copy(p_r.at[0].at[pl.ds(base, nrows)],
                            buf0.at[pl.ds(0, nrows)])
            pltpu.sync_copy(p_r.at[1].at[pl.ds(base, nrows)],
                            buf1.at[pl.ds(0, nrows)])
            bias = [bias_v[pl.ds(16 * k, 16)]
                    for k in range(NCLS_PAD // 16)]

            def add_body(r, carry):
                for k in range(NCLS_PAD // 16):
                    col = 16 * k
                    buf0[r, pl.ds(col, 16)] = (buf0[r, pl.ds(col, 16)]
                                               + buf1[r, pl.ds(col, 16)]
                                               + bias[k])
                return carry

            lax.fori_loop(0, nrows, add_body, 0)
            pltpu.sync_copy(buf0.at[pl.ds(0, nrows)],
                            out.at[pl.ds(base, nrows)])

        @pl.when(w < 31)
        def _():
            do(CROWS)

        @pl.when(w == 31)
        def _():
            do(CLAST)

    return comb(p, b2p)


def kernel(x, edge_index, W1, b1, gamma, beta, W2, b2):
    e = edge_index.shape[1]
    g_total = -(-e // GROUP)
    e_pad = g_total * GROUP
    if e_pad == e:
        src = edge_index[0].reshape(g_total, GROUP)
        dst = edge_index[1].reshape(g_total, GROUP)
    else:
        pad_idx = jnp.arange(e_pad - e, dtype=jnp.int32)
        dummy_src = pad_idx % N
        dummy_dst = N + pad_idx % (ACC_ROWS - N)
        src = jnp.concatenate([edge_index[0], dummy_src]).reshape(
            g_total, GROUP)
        dst = jnp.concatenate([edge_index[1], dummy_dst]).reshape(
            g_total, GROUP)

    w2p = jnp.pad(W2, ((0, 0), (0, NCLS_PAD - NCLASS)))

    agg1 = _make_sc_aggregate(NFEAT, g_total)(x, src, dst)

    grid = (N // _BM,)
    cs, gmat = pl.pallas_call(
        _tca_body,
        grid=grid,
        in_specs=[
            pl.BlockSpec((NC, _BM, NFEAT), lambda i: (0, i, 0)),
        ],
        out_specs=[
            pl.BlockSpec((1, NFEAT), lambda i: (0, 0)),
            pl.BlockSpec((NFEAT, NFEAT), lambda i: (0, 0)),
        ],
        out_shape=[
            jax.ShapeDtypeStruct((1, NFEAT), jnp.float32),
            jax.ShapeDtypeStruct((NFEAT, NFEAT), jnp.float32),
        ],
    )(agg1)

    scale, shift = pl.pallas_call(
        _tcs_body,
        out_shape=[
            jax.ShapeDtypeStruct((1, NHID), jnp.float32),
            jax.ShapeDtypeStruct((1, NHID), jnp.float32),
        ],
    )(cs, gmat, W1, gamma.reshape(1, NHID), beta.reshape(1, NHID))

    y = pl.pallas_call(
        _tcc_body,
        grid=grid,
        in_specs=[
            pl.BlockSpec((NC, _BM, NFEAT), lambda i: (0, i, 0)),
            pl.BlockSpec((NFEAT, NHID), lambda i: (0, 0)),
            pl.BlockSpec((1, NHID), lambda i: (0, 0)),
            pl.BlockSpec((1, NHID), lambda i: (0, 0)),
            pl.BlockSpec((NHID, NCLS_PAD), lambda i: (0, 0)),
        ],
        out_specs=pl.BlockSpec((_BM, NCLS_PAD), lambda i: (i, 0)),
        out_shape=jax.ShapeDtypeStruct((N, NCLS_PAD), jnp.float32),
    )(agg1, W1, scale, shift, w2p)

    agg2 = _make_sc_aggregate(NCLS_PAD, g_total)(y, src, dst)

    b2p = jnp.pad(b2, (0, NCLS_PAD - NCLASS))
    out64 = _sc_combine(agg2, b2p)

    return out64[:, :NCLASS]


# speedup vs baseline: 1.1818x; 1.0560x over previous
import functools

import jax
import jax.numpy as jnp
from jax import lax
from jax.experimental import pallas as pl
from jax.experimental.pallas import tpu as pltpu
from jax.experimental.pallas import tpu_sc as plsc

N = 10000
NFEAT = 128
NHID = 256
NCLASS = 40
NCLS_PAD = 64
EPS = 1e-5

NC = 2
NS = 16
GROUP = 128
ACC_ROWS = 10240
ZR = ACC_ROWS // NS
ZR_MAIN = N - (NS - 1) * ZR
CHUNK = 32


def _make_sc_aggregate(d: int, g: int):
    mesh = plsc.VectorSubcoreMesh(core_axis_name="c", subcore_axis_name="s")

    nbuf = 2 if d == 128 else 4

    @functools.partial(
        pl.kernel,
        out_type=jax.ShapeDtypeStruct((NC, N, d), jnp.float32),
        mesh=mesh,
        compiler_params=pltpu.CompilerParams(use_tc_tiling_on_sc=False),
        scratch_types=(
            [pltpu.VMEM((CHUNK, 2, GROUP), jnp.int32)]
            + [pltpu.VMEM((GROUP, d), jnp.float32) for _ in range(nbuf)]
            + [pltpu.VMEM_SHARED((ACC_ROWS, d), jnp.float32)]
            + [pltpu.SemaphoreType.DMA for _ in range(nbuf)]
        ),
    )
    def agg(table, er, out, idx_v, *rest):
        rows = rest[:nbuf]
        acc = rest[nbuf]
        sems = rest[nbuf + 1:]
        c = lax.axis_index("c")
        s = lax.axis_index("s")
        w = s * NC + c

        def zbody(i, carry):
            for k in range(d // 16):
                rows[0][i, pl.ds(16 * k, 16)] = jnp.zeros((16,), jnp.float32)
            return carry

        lax.fori_loop(0, GROUP, zbody, 0)
        for r in range(ZR // GROUP):
            pltpu.sync_copy(rows[0], acc.at[pl.ds(s * ZR + r * GROUP, GROUP)])
        plsc.subcore_barrier()

        def run(gbase, gw):
            for off in range(0, gw, CHUNK):
                cs = min(CHUNK, gw - off)
                pltpu.sync_copy(er.at[pl.ds(gbase + off, cs)],
                                idx_v.at[pl.ds(0, cs)])
                for k in range(min(nbuf, cs)):
                    pltpu.async_copy(table.at[idx_v.at[k, 0]], rows[k],
                                     sems[k])
                nfull = cs // nbuf

                def body(i, carry, cs=cs):
                    j0 = i * nbuf
                    for k in range(nbuf):
                        j = j0 + k
                        pltpu.make_async_copy(table.at[idx_v.at[0, 0]],
                                              rows[k], sems[k]).wait()
                        pltpu.sync_copy(rows[k], acc.at[idx_v.at[j, 1]],
                                        add=True)

                        @pl.when(j + nbuf < cs)
                        def _(k=k, j=j):
                            pltpu.async_copy(table.at[idx_v.at[j + nbuf, 0]],
                                             rows[k], sems[k])
                    return carry

                if nfull:
                    lax.fori_loop(0, nfull, body, 0)
                for k in range(cs % nbuf):
                    j = nfull * nbuf + k
                    pltpu.make_async_copy(table.at[idx_v.at[0, 0]],
                                          rows[k], sems[k]).wait()
                    pltpu.sync_copy(rows[k], acc.at[idx_v.at[j, 1]],
                                    add=True)

        nw = NC * NS
        rem = g % nw
        gq = g // nw
        gbase = w * gq + jnp.minimum(w, rem)
        if rem:
            @pl.when(w < rem)
            def _():
                run(gbase, gq + 1)

            @pl.when(w >= rem)
            def _():
                run(gbase, gq)
        else:
            run(gbase, gq)

        plsc.subcore_barrier()

        base = s * ZR
        pltpu.sync_copy(acc.at[pl.ds(base, ZR_MAIN)],
                        out.at[c].at[pl.ds(base, ZR_MAIN)])

        @pl.when(s < NS - 1)
        def _():
            pltpu.sync_copy(acc.at[pl.ds(base + ZR_MAIN, ZR - ZR_MAIN)],
                            out.at[c].at[pl.ds(base + ZR_MAIN, ZR - ZR_MAIN)])

    return agg


_BM = 2000


def _tca_body(ap, cs_ref, g_ref):
    a = ap[0] + ap[1]

    @pl.when(pl.program_id(0) == 0)
    def _():
        cs_ref[...] = jnp.zeros_like(cs_ref)
        g_ref[...] = jnp.zeros_like(g_ref)

    cs_ref[...] += jnp.sum(a, axis=0, keepdims=True)
    g_ref[...] += lax.dot_general(a, a, (((0,), (0,)), ((), ())),
                                  preferred_element_type=jnp.float32)


def _tcs_body(cs, g, w1, gamma, beta, scale_ref, shift_ref):
    w1v = w1[...]
    mean0 = jnp.dot(cs[...], w1v, preferred_element_type=jnp.float32) * (
        1.0 / N)
    gw = jnp.dot(g[...], w1v, preferred_element_type=jnp.float32)
    ssq = jnp.sum(w1v * gw, axis=0, keepdims=True)
    var = ssq * (1.0 / N) - mean0 * mean0
    scale = gamma[...] * lax.rsqrt(var + EPS)
    scale_ref[...] = scale
    shift_ref[...] = beta[...] - mean0 * scale


def _tcc_body(ap, w1, scale, shift, w2, y_ref):
    a = ap[0] + ap[1]
    h0 = jnp.dot(a, w1[...], preferred_element_type=jnp.float32)
    hr = jnp.maximum(h0 * scale[...] + shift[...], 0.0)
    y_ref[...] = jnp.dot(hr, w2[...], preferred_element_type=jnp.float32)


CROWS = 313
CLAST = N - 31 * CROWS


def _sc_combine(p, b2p):
    mesh = plsc.VectorSubcoreMesh(core_axis_name="c", subcore_axis_name="s")

    @functools.partial(
        pl.kernel,
        out_type=jax.ShapeDtypeStruct((N, NCLS_PAD), jnp.float32),
        mesh=mesh,
        compiler_params=pltpu.CompilerParams(use_tc_tiling_on_sc=False),
        scratch_types=[
            pltpu.VMEM((CROWS, NCLS_PAD), jnp.float32),
            pltpu.VMEM((CROWS, NCLS_PAD), jnp.float32),
            pltpu.VMEM((NCLS_PAD,), jnp.float32),
        ],
    )
    def comb(p_r, b2_r, out, buf0, buf1, bias_v):
        c = lax.axis_index("c")
        sx = lax.axis_index("s")
        w = sx * NC + c
        base = w * CROWS
        pltpu.sync_copy(b2_r, bias_v)

        def do(nrows):
            pltpu.sync_copy(p_r.at[0].at[pl.ds(base, nrows)],
                            buf0.at[pl.ds(0, nrows)])
            pltpu.sync_copy(p_r.at[1].at[pl.ds(base, nrows)],
                            buf1.at[pl.ds(0, nrows)])
            bias = [bias_v[pl.ds(16 * k, 16)]
                    for k in range(NCLS_PAD // 16)]

            def add_body(r, carry):
                for k in range(NCLS_PAD // 16):
                    col = 16 * k
                    buf0[r, pl.ds(col, 16)] = (buf0[r, pl.ds(col, 16)]
                                               + buf1[r, pl.ds(col, 16)]
                                               + bias[k])
                return carry

            lax.fori_loop(0, nrows, add_body, 0)
            pltpu.sync_copy(buf0.at[pl.ds(0, nrows)],
                            out.at[pl.ds(base, nrows)])

        @pl.when(w < 31)
        def _():
            do(CROWS)

        @pl.when(w == 31)
        def _():
            do(CLAST)

    return comb(p, b2p)


def kernel(x, edge_index, W1, b1, gamma, beta, W2, b2):
    e = edge_index.shape[1]
    g_total = -(-e // GROUP)
    e_pad = g_total * GROUP
    if e_pad != e:
        pad_idx = jnp.arange(e_pad - e, dtype=jnp.int32)
        dummy = jnp.stack([pad_idx % N, N + pad_idx % (ACC_ROWS - N)])
        edge_index = jnp.concatenate([edge_index, dummy], axis=1)
    er = jnp.stack([edge_index[0].reshape(g_total, GROUP),
                    edge_index[1].reshape(g_total, GROUP)], axis=1)

    w2p = jnp.pad(W2, ((0, 0), (0, NCLS_PAD - NCLASS)))

    agg1 = _make_sc_aggregate(NFEAT, g_total)(x, er)

    grid = (N // _BM,)
    cs, gmat = pl.pallas_call(
        _tca_body,
        grid=grid,
        in_specs=[
            pl.BlockSpec((NC, _BM, NFEAT), lambda i: (0, i, 0)),
        ],
        out_specs=[
            pl.BlockSpec((1, NFEAT), lambda i: (0, 0)),
            pl.BlockSpec((NFEAT, NFEAT), lambda i: (0, 0)),
        ],
        out_shape=[
            jax.ShapeDtypeStruct((1, NFEAT), jnp.float32),
            jax.ShapeDtypeStruct((NFEAT, NFEAT), jnp.float32),
        ],
    )(agg1)

    scale, shift = pl.pallas_call(
        _tcs_body,
        out_shape=[
            jax.ShapeDtypeStruct((1, NHID), jnp.float32),
            jax.ShapeDtypeStruct((1, NHID), jnp.float32),
        ],
    )(cs, gmat, W1, gamma.reshape(1, NHID), beta.reshape(1, NHID))

    y = pl.pallas_call(
        _tcc_body,
        grid=grid,
        in_specs=[
            pl.BlockSpec((NC, _BM, NFEAT), lambda i: (0, i, 0)),
            pl.BlockSpec((NFEAT, NHID), lambda i: (0, 0)),
            pl.BlockSpec((1, NHID), lambda i: (0, 0)),
            pl.BlockSpec((1, NHID), lambda i: (0, 0)),
            pl.BlockSpec((NHID, NCLS_PAD), lambda i: (0, 0)),
        ],
        out_specs=pl.BlockSpec((_BM, NCLS_PAD), lambda i: (i, 0)),
        out_shape=jax.ShapeDtypeStruct((N, NCLS_PAD), jnp.float32),
    )(agg1, W1, scale, shift, w2p)

    agg2 = _make_sc_aggregate(NCLS_PAD, g_total)(y, er)

    b2p = jnp.pad(b2, (0, NCLS_PAD - NCLASS))
    out64 = _sc_combine(agg2, b2p)

    return out64[:, :NCLASS]
